# baseline (speedup 1.0000x reference)
import functools

import jax
import jax.numpy as jnp
from jax import lax
from jax.experimental import pallas as pl
from jax.experimental.pallas import tpu as pltpu
from jax.experimental.pallas import tpu_sc as plsc

_INFO = plsc.get_sparse_core_info()
_NC = _INFO.num_cores
_NS = _INFO.num_subcores
_NW = _NC * _NS

_TPW = 244
_BC = 256
_V_ALIGNED = _NW * _TPW * 128


@functools.cache
def _make_sc_transpose(v, d, v_pad):
    nblk = (_TPW * 128) // _BC
    mesh = plsc.VectorSubcoreMesh(core_axis_name="c", subcore_axis_name="s")

    @functools.partial(
        pl.kernel,
        mesh=mesh,
        out_type=jax.ShapeDtypeStruct((v_pad // 2, 2 * d), jnp.float32),
        compiler_params=pltpu.CompilerParams(needs_layout_passes=False),
        scratch_types=[
            pltpu.VMEM((d, _BC), jnp.float32),
            pltpu.VMEM((d, _BC), jnp.float32),
            pltpu.VMEM((_BC // 2, 2 * d), jnp.float32),
            pltpu.VMEM((_BC // 2, 2 * d), jnp.float32),
            pltpu.SemaphoreType.DMA,
            pltpu.SemaphoreType.DMA,
            pltpu.SemaphoreType.DMA,
            pltpu.SemaphoreType.DMA,
        ],
    )
    def transpose(at_hbm, out_hbm, in0, in1, ob0, ob1, gi0, gi1, go0, go1):
        wid = lax.axis_index("s") * _NC + lax.axis_index("c")
        col0 = pl.multiple_of(wid * (_TPW * 128), 128)
        ins = (in0, in1)
        obs = (ob0, ob1)
        gis = (gi0, gi1)
        gos = (go0, go1)

        def in_src(k):
            return at_hbm.at[:, pl.ds(pl.multiple_of(col0 + k * _BC, 128),
                                      _BC)]

        def out_dst(k):
            row0 = pl.multiple_of((col0 + k * _BC) // 2, 8)
            return out_hbm.at[pl.ds(row0, _BC // 2)]

        def compute(ibuf, obuf):
            def one(p):
                cols = jnp.full((16,), 2 * p, jnp.int32)
                for e in range(4):
                    rows = lax.iota(jnp.int32, 16) + e * 16
                    va = plsc.load_gather(ibuf, [rows, cols])
                    vb = plsc.load_gather(ibuf, [rows, cols + 1])
                    obuf[p, pl.ds(e * 16, 16)] = va
                    obuf[p, pl.ds(d + e * 16, 16)] = vb

            pl.loop(0, _BC // 2)(one)

        pltpu.async_copy(in_src(0), ins[0], gis[0])
        pltpu.async_copy(in_src(1), ins[1], gis[1])

        def step(k):
            for half in range(2):
                kk = k + half
                ibuf, obuf = ins[half], obs[half]
                pltpu.make_async_copy(in_src(kk), ibuf, gis[half]).wait()

                @pl.when(kk >= 2)
                def _():
                    pltpu.make_async_copy(obuf, out_dst(kk - 2),
                                          gos[half]).wait()

                compute(ibuf, obuf)

                @pl.when(kk + 2 < nblk)
                def _():
                    pltpu.async_copy(in_src(kk + 2), ibuf, gis[half])

                pltpu.async_copy(obuf, out_dst(kk), gos[half])

        pl.loop(0, nblk, step=2)(step)
        pltpu.make_async_copy(obs[0], out_dst(nblk - 2), gos[0]).wait()
        pltpu.make_async_copy(obs[1], out_dst(nblk - 1), gos[1]).wait()

    return transpose


@functools.cache
def _make_tc_tail(v, d, v_pad):
    dims = (((1,), (1,)), ((), ()))

    def body(prev_ref, a_ref, out_ref):
        del prev_ref
        x = a_ref[...]
        r = lax.broadcasted_iota(jnp.int32, (512, 1024), 1)
        c = lax.broadcasted_iota(jnp.int32, (512, 1024), 0)
        se = (r == 2 * c).astype(jnp.float32)
        so = (r == 2 * c + 1).astype(jnp.float32)
        t0 = lax.dot_general(se, x, dims,
                             preferred_element_type=jnp.float32)
        t1 = lax.dot_general(so, x, dims,
                             preferred_element_type=jnp.float32)
        out_ref[...] = jnp.concatenate([t0, t1], axis=1)

    return pl.pallas_call(
        body,
        grid=(1,),
        in_specs=[
            pl.BlockSpec(memory_space=pl.ANY),
            pl.BlockSpec((d, 1024), lambda i: (0, _V_ALIGNED // 1024)),
        ],
        out_specs=pl.BlockSpec((512, 2 * d),
                               lambda i: (_V_ALIGNED // 1024, 0)),
        out_shape=jax.ShapeDtypeStruct((v_pad // 2, 2 * d), jnp.float32),
        input_output_aliases={0: 0},
    )


@functools.cache
def _make_gather(n_rows: int, d: int, chunk: int, v_pad: int):
    per_w = n_rows // _NW
    nchunk = per_w // chunk
    mesh = plsc.VectorSubcoreMesh(core_axis_name="c", subcore_axis_name="s")

    @functools.partial(
        pl.kernel,
        mesh=mesh,
        out_type=jax.ShapeDtypeStruct((n_rows, d), jnp.float32),
        compiler_params=pltpu.CompilerParams(use_tc_tiling_on_sc=False),
        scratch_types=(
            [pltpu.VMEM((chunk,), jnp.int32) for _ in range(nchunk)]
            + [
                pltpu.VMEM((chunk, d), jnp.float32),
                pltpu.VMEM((chunk, d), jnp.float32),
                pltpu.SemaphoreType.DMA,
                pltpu.SemaphoreType.DMA,
                pltpu.SemaphoreType.DMA,
                pltpu.SemaphoreType.DMA,
            ]
        ),
    )
    def gather(table_hbm, idx_hbm, out_hbm, *scratch):
        idx_vs = scratch[:nchunk]
        buf0, buf1, g0, g1, o0, o1 = scratch[nchunk:]
        wid = lax.axis_index("s") * _NC + lax.axis_index("c")
        base = wid * per_w
        for i in range(nchunk):
            pltpu.sync_copy(idx_hbm.at[wid, i], idx_vs[i])
        bufs = (buf0, buf1)
        gsems = (g0, g1)
        osems = (o0, o1)
        gathers = [None] * nchunk
        outs = [None] * nchunk
        gathers[0] = pltpu.async_copy(table_hbm.at[idx_vs[0]], bufs[0],
                                      gsems[0])
        for i in range(nchunk):
            cur = i % 2
            nxt = (i + 1) % 2
            if i + 1 < nchunk:
                if i >= 1:
                    outs[i - 1].wait()
                gathers[i + 1] = pltpu.async_copy(
                    table_hbm.at[idx_vs[i + 1]], bufs[nxt], gsems[nxt])
            gathers[i].wait()
            outs[i] = pltpu.async_copy(
                bufs[cur], out_hbm.at[pl.ds(base + i * chunk, chunk)],
                osems[cur])
        if nchunk >= 2:
            outs[nchunk - 2].wait()
        outs[nchunk - 1].wait()

    return gather


def kernel(token_ids, std_embed, q6_basis, hamming_scale):
    b, t = token_ids.shape
    v, d = std_embed.shape
    n_rows = b * t
    chunk = 640
    per_w = n_rows // _NW
    v_pad = 2 * (_V_ALIGNED // 2 + 512)
    at = std_embed.T
    table128 = _make_sc_transpose(v, d, v_pad)(at)
    table128 = _make_tc_tail(v, d, v_pad)(table128, at)
    table = table128.reshape(v_pad, d)
    idx = token_ids.astype(jnp.int32).reshape(_NW, per_w // chunk, chunk)
    out = _make_gather(n_rows, d, chunk, v_pad)(table, idx)
    return (out.reshape(b, t, d), None)


# speedup vs baseline: 2.0872x; 2.0872x over previous
import functools

import jax
import jax.numpy as jnp
from jax import lax
from jax.experimental import pallas as pl
from jax.experimental.pallas import tpu as pltpu
from jax.experimental.pallas import tpu_sc as plsc

_INFO = plsc.get_sparse_core_info()
_NC = _INFO.num_cores
_NS = _INFO.num_subcores
_NW = _NC * _NS


@functools.cache
def _make_transpose(v_pad, d, bc):
    nblk = v_pad // (2 * bc)

    def body(a_ref, out_ref):
        x = a_ref[...]
        out_ref[...] = jnp.concatenate(
            [x[:, :bc].T, x[:, bc:].T], axis=1)

    return pl.pallas_call(
        body,
        grid=(nblk,),
        in_specs=[pl.BlockSpec((d, 2 * bc), lambda i: (0, i))],
        out_specs=pl.BlockSpec((bc, 2 * d), lambda i: (i, 0)),
        out_shape=jax.ShapeDtypeStruct((v_pad // 2, 2 * d), jnp.float32),
    )


@functools.cache
def _make_gather(n_rows: int, d: int, chunk: int):
    per_w = n_rows // _NW
    nchunk = per_w // chunk
    mesh = plsc.VectorSubcoreMesh(core_axis_name="c", subcore_axis_name="s")

    @functools.partial(
        pl.kernel,
        mesh=mesh,
        out_type=jax.ShapeDtypeStruct((n_rows, d), jnp.float32),
        compiler_params=pltpu.CompilerParams(use_tc_tiling_on_sc=False),
        scratch_types=(
            [pltpu.VMEM((chunk,), jnp.int32) for _ in range(nchunk)]
            + [
                pltpu.VMEM((chunk, d), jnp.float32),
                pltpu.VMEM((chunk, d), jnp.float32),
                pltpu.SemaphoreType.DMA,
                pltpu.SemaphoreType.DMA,
                pltpu.SemaphoreType.DMA,
                pltpu.SemaphoreType.DMA,
            ]
        ),
    )
    def gather(table_hbm, idx_hbm, out_hbm, *scratch):
        idx_vs = scratch[:nchunk]
        buf0, buf1, g0, g1, o0, o1 = scratch[nchunk:]
        wid = lax.axis_index("s") * _NC + lax.axis_index("c")
        base = wid * per_w
        for i in range(nchunk):
            pltpu.sync_copy(idx_hbm.at[wid, i], idx_vs[i])
        bufs = (buf0, buf1)
        gsems = (g0, g1)
        osems = (o0, o1)
        gathers = [None] * nchunk
        outs = [None] * nchunk
        gathers[0] = pltpu.async_copy(table_hbm.at[idx_vs[0]], bufs[0],
                                      gsems[0])
        for i in range(nchunk):
            cur = i % 2
            nxt = (i + 1) % 2
            if i + 1 < nchunk:
                if i >= 1:
                    outs[i - 1].wait()
                gathers[i + 1] = pltpu.async_copy(
                    table_hbm.at[idx_vs[i + 1]], bufs[nxt], gsems[nxt])
            gathers[i].wait()
            outs[i] = pltpu.async_copy(
                bufs[cur], out_hbm.at[pl.ds(base + i * chunk, chunk)],
                osems[cur])
        if nchunk >= 2:
            outs[nchunk - 2].wait()
        outs[nchunk - 1].wait()

    return gather


def kernel(token_ids, std_embed, q6_basis, hamming_scale):
    b, t = token_ids.shape
    v, d = std_embed.shape
    n_rows = b * t
    chunk = 640
    per_w = n_rows // _NW
    bc = 1024
    v_pad = -(-v // (2 * bc)) * (2 * bc)
    table128 = _make_transpose(v_pad, d, bc)(std_embed.T)
    table = table128.reshape(v_pad, d)
    flat = token_ids.reshape(-1).astype(jnp.int32)
    blk = flat >> 11
    j = flat & (2 * bc - 1)
    remapped = ((blk * bc + (j & (bc - 1))) << 1) | (j >> 10)
    idx = remapped.reshape(_NW, per_w // chunk, chunk)
    out = _make_gather(n_rows, d, chunk)(table, idx)
    return (out.reshape(b, t, d), None)


# speedup vs baseline: 2.5708x; 1.2317x over previous
import functools

import jax
import jax.numpy as jnp
from jax import lax
from jax.experimental import pallas as pl
from jax.experimental.pallas import tpu as pltpu
from jax.experimental.pallas import tpu_sc as plsc

_INFO = plsc.get_sparse_core_info()
_NC = _INFO.num_cores
_NS = _INFO.num_subcores
_NW = _NC * _NS


@functools.cache
def _make_transpose(v_pad, d, bc):
    nblk = v_pad // (2 * bc)

    def body(a_ref, out_ref):
        x = a_ref[...]
        out_ref[...] = jnp.concatenate(
            [x[:, :bc].T, x[:, bc:].T], axis=1)

    return pl.pallas_call(
        body,
        grid=(nblk,),
        in_specs=[pl.BlockSpec((d, 2 * bc), lambda i: (0, i))],
        out_specs=pl.BlockSpec((bc, 2 * d), lambda i: (i, 0)),
        out_shape=jax.ShapeDtypeStruct((v_pad // 2, 2 * d), jnp.float32),
    )


@functools.cache
def _make_gather(n_rows: int, d: int, chunk: int):
    per_w = n_rows // _NW
    nchunk = per_w // chunk
    mesh = plsc.VectorSubcoreMesh(core_axis_name="c", subcore_axis_name="s")

    @functools.partial(
        pl.kernel,
        mesh=mesh,
        out_type=jax.ShapeDtypeStruct((n_rows, d), jnp.float32),
        compiler_params=pltpu.CompilerParams(use_tc_tiling_on_sc=False),
        scratch_types=(
            [pltpu.VMEM((chunk,), jnp.int32) for _ in range(nchunk)]
            + [
                pltpu.VMEM((chunk, d), jnp.float32),
                pltpu.VMEM((chunk, d), jnp.float32),
                pltpu.SemaphoreType.DMA,
                pltpu.SemaphoreType.DMA,
                pltpu.SemaphoreType.DMA,
                pltpu.SemaphoreType.DMA,
            ]
        ),
    )
    def gather(table_hbm, idx_hbm, out_hbm, *scratch):
        idx_vs = scratch[:nchunk]
        buf0, buf1, g0, g1, o0, o1 = scratch[nchunk:]
        wid = lax.axis_index("s") * _NC + lax.axis_index("c")
        base = wid * per_w
        for i in range(nchunk):
            pltpu.sync_copy(idx_hbm.at[wid, i], idx_vs[i])
        bufs = (buf0, buf1)
        gsems = (g0, g1)
        osems = (o0, o1)
        gathers = [None] * nchunk
        outs = [None] * nchunk
        gathers[0] = pltpu.async_copy(table_hbm.at[idx_vs[0]], bufs[0],
                                      gsems[0])
        for i in range(nchunk):
            cur = i % 2
            nxt = (i + 1) % 2
            if i + 1 < nchunk:
                if i >= 1:
                    outs[i - 1].wait()
                gathers[i + 1] = pltpu.async_copy(
                    table_hbm.at[idx_vs[i + 1]], bufs[nxt], gsems[nxt])
            gathers[i].wait()
            outs[i] = pltpu.async_copy(
                bufs[cur], out_hbm.at[pl.ds(base + i * chunk, chunk)],
                osems[cur])
        if nchunk >= 2:
            outs[nchunk - 2].wait()
        outs[nchunk - 1].wait()

    return gather


def kernel(token_ids, std_embed, q6_basis, hamming_scale):
    b, t = token_ids.shape
    v, d = std_embed.shape
    n_rows = b * t
    chunk = 640
    per_w = n_rows // _NW
    bc = 2048
    v_pad = -(-v // (2 * bc)) * (2 * bc)
    table128 = _make_transpose(v_pad, d, bc)(std_embed.T)
    table = table128.reshape(v_pad, d)
    flat = token_ids.reshape(-1).astype(jnp.int32)
    sh = bc.bit_length() - 1
    blk = flat >> (sh + 1)
    j = flat & (2 * bc - 1)
    remapped = ((blk * bc + (j & (bc - 1))) << 1) | (j >> sh)
    idx = remapped.reshape(_NW, per_w // chunk, chunk)
    out = _make_gather(n_rows, d, chunk)(table, idx)
    return (out.reshape(b, t, d), None)
